# baseline (speedup 1.0000x reference)
import functools

import jax
import jax.numpy as jnp
from jax import lax
from jax.experimental import pallas as pl
from jax.experimental.pallas import tpu as pltpu
from jax.experimental.pallas import tpu_sc as plsc

VOCAB = 100000
EMBED = 64
MAX_POS = 200
BATCH = 4096
SEQ = 200

NC = 2
NS = 16
NW = NC * NS
LANES = 16

BGRP = BATCH // NW
NGRP = EMBED // LANES
DBLK = 2
TPAD = BGRP + 1
NSUP = MAX_POS // (2 * DBLK)


def _emb_body(table_hbm, pos_hbm, ids_hbm, out_hbm,
              pos_v, ids_v, rows0, rows1, tr0, tr1,
              sg0, sg1, so0, so1):
    cid = lax.axis_index("c")
    sid = lax.axis_index("s")
    wid = sid * NC + cid
    b0 = wid * BGRP

    pltpu.sync_copy(pos_hbm, pos_v)

    iotas_e = [lax.iota(jnp.int32, LANES) + (LANES * j) for j in range(NGRP)]
    iotas_hi = [v // 8 for v in iotas_e]
    iotas_lo = [v % 8 for v in iotas_e]

    def gathers(l0, rows_ref, sem):
        for d in range(DBLK):
            pltpu.async_copy(
                table_hbm.at[ids_v.at[l0 + d]],
                rows_ref.at[pl.ds(d * BGRP, BGRP)],
                sem,
            )

    def wait_gathers(rows_ref, sem):
        pltpu.make_async_copy(
            table_hbm.at[pl.ds(0, DBLK * BGRP)], rows_ref, sem
        ).wait()

    def wait_outs(tr_ref, sem):
        for d in range(DBLK):
            pltpu.make_async_copy(
                tr_ref.at[d, :, :, pl.ds(0, BGRP)],
                out_hbm.at[0, :, wid],
                sem,
            ).wait()

    def transpose_add(l, rows_ref, rbase, tr_view):
        pjs = [pos_v[l, pl.ds(LANES * j, LANES)] for j in range(NGRP)]

        @plsc.parallel_loop(0, BGRP, 1, unroll=8)
        def _(b):
            bvec = jnp.full((LANES,), b, jnp.int32)
            for j in range(NGRP):
                v = rows_ref[rbase + b, pl.ds(LANES * j, LANES)] + pjs[j]
                plsc.store_scatter(tr_view, [iotas_hi[j], iotas_lo[j], bvec], v)

    def outs(tr_ref, l0, sem):
        for d in range(DBLK):
            pltpu.async_copy(
                tr_ref.at[d, :, :, pl.ds(0, BGRP)],
                out_hbm.at[l0 + d, :, wid],
                sem,
            )

    pltpu.sync_copy(ids_hbm.at[:, pl.ds(b0, BGRP)], ids_v)
    gathers(0, rows0, sg0)

    def superstep(s, carry):
        l0 = (2 * DBLK) * s

        gathers(l0 + DBLK, rows1, sg1)
        wait_gathers(rows0, sg0)

        @pl.when(s > 0)
        def _():
            wait_outs(tr0, so0)

        transpose_add(l0, rows0, 0, tr0.at[0])
        transpose_add(l0 + 1, rows0, BGRP, tr0.at[1])
        outs(tr0, l0, so0)

        @pl.when(s < NSUP - 1)
        def _():
            gathers(l0 + 2 * DBLK, rows0, sg0)

        wait_gathers(rows1, sg1)

        @pl.when(s > 0)
        def _():
            wait_outs(tr1, so1)

        transpose_add(l0 + DBLK, rows1, 0, tr1.at[0])
        transpose_add(l0 + DBLK + 1, rows1, BGRP, tr1.at[1])
        outs(tr1, l0 + DBLK, so1)
        return carry

    lax.fori_loop(0, NSUP, superstep, 0)
    wait_outs(tr0, so0)
    wait_outs(tr1, so1)


@jax.jit
def _emb(table, pos2d, ids_t):
    mesh = plsc.VectorSubcoreMesh(core_axis_name="c", subcore_axis_name="s")
    return pl.kernel(
        _emb_body,
        out_type=jax.ShapeDtypeStruct((MAX_POS, EMBED // 8, NW, 8, BGRP), jnp.float32),
        mesh=mesh,
        scratch_types=[
            pltpu.VMEM((MAX_POS, EMBED), jnp.float32),
            pltpu.VMEM((MAX_POS, BGRP), jnp.int32),
            pltpu.VMEM((DBLK * BGRP, EMBED), jnp.float32),
            pltpu.VMEM((DBLK * BGRP, EMBED), jnp.float32),
            pltpu.VMEM((DBLK, EMBED // 8, 8, TPAD), jnp.float32),
            pltpu.VMEM((DBLK, EMBED // 8, 8, TPAD), jnp.float32),
            pltpu.SemaphoreType.DMA,
            pltpu.SemaphoreType.DMA,
            pltpu.SemaphoreType.DMA,
            pltpu.SemaphoreType.DMA,
        ],
        compiler_params=pltpu.CompilerParams(
            use_tc_tiling_on_sc=False, needs_layout_passes=False
        ),
    )(table, pos2d, ids_t)


def kernel(embedding_table, position_embeds, input_ids):
    ids_t = jnp.transpose(input_ids.astype(jnp.int32))
    pos2d = position_embeds.reshape(MAX_POS, EMBED)
    outp = _emb(embedding_table, pos2d, ids_t)
    return outp.transpose(2, 4, 0, 1, 3).reshape(BATCH, SEQ, EMBED)


# speedup vs baseline: 1.0512x; 1.0512x over previous
import functools

import jax
import jax.numpy as jnp
from jax import lax
from jax.experimental import pallas as pl
from jax.experimental.pallas import tpu as pltpu
from jax.experimental.pallas import tpu_sc as plsc

VOCAB = 100000
EMBED = 64
MAX_POS = 200
BATCH = 4096
SEQ = 200

NC = 2
NS = 16
NW = NC * NS
LANES = 16

BGRP = BATCH // NW
NGRP = EMBED // LANES
DBLK = 2
TPAD = BGRP + 1
NSUP = MAX_POS // (2 * DBLK)


def _emb_body(table_hbm, pos_hbm, ids_hbm, out_hbm,
              pos_v, idx0, idx1, rows0, rows1, tr0, tr1,
              sg0, sg1, so0, so1, si0, si1):
    cid = lax.axis_index("c")
    sid = lax.axis_index("s")
    wid = sid * NC + cid
    b0 = wid * BGRP

    pltpu.sync_copy(pos_hbm, pos_v)

    iotas_e = [lax.iota(jnp.int32, LANES) + (LANES * j) for j in range(NGRP)]
    iotas_hi = [v // 8 for v in iotas_e]
    iotas_lo = [v % 8 for v in iotas_e]

    def stage(l0, idx_ref, sem):
        pltpu.async_copy(
            ids_hbm.at[pl.ds(l0, DBLK), pl.ds(b0, BGRP)], idx_ref, sem
        )

    def wait_stage(idx_ref, sem):
        pltpu.make_async_copy(
            ids_hbm.at[pl.ds(0, DBLK), pl.ds(b0, BGRP)], idx_ref, sem
        ).wait()

    def gathers(idx_ref, rows_ref, sem):
        for d in range(DBLK):
            pltpu.async_copy(
                table_hbm.at[idx_ref.at[d]],
                rows_ref.at[pl.ds(d * BGRP, BGRP)],
                sem,
            )

    def wait_gather1(sem):
        pltpu.make_async_copy(
            table_hbm.at[pl.ds(0, BGRP)],
            rows0.at[pl.ds(0, BGRP)],
            sem,
        ).wait()

    def wait_outs(tr_ref, sem):
        for d in range(DBLK):
            pltpu.make_async_copy(
                tr_ref.at[d, :, :, pl.ds(0, BGRP)],
                out_hbm.at[0, :, wid],
                sem,
            ).wait()

    def transpose_add(l, rows_ref, rbase, tr_view):
        pjs = [pos_v[l, pl.ds(LANES * j, LANES)] for j in range(NGRP)]

        @plsc.parallel_loop(0, BGRP, 1, unroll=8)
        def _(b):
            bvec = jnp.full((LANES,), b, jnp.int32)
            for j in range(NGRP):
                v = rows_ref[rbase + b, pl.ds(LANES * j, LANES)] + pjs[j]
                plsc.store_scatter(tr_view, [iotas_hi[j], iotas_lo[j], bvec], v)

    def outs(tr_ref, l0, sem):
        for d in range(DBLK):
            pltpu.async_copy(
                tr_ref.at[d, :, :, pl.ds(0, BGRP)],
                out_hbm.at[l0 + d, :, wid],
                sem,
            )

    pltpu.sync_copy(ids_hbm.at[pl.ds(0, DBLK), pl.ds(b0, BGRP)], idx0)
    gathers(idx0, rows0, sg0)
    stage(DBLK, idx1, si1)

    def superstep(s, carry):
        l0 = (2 * DBLK) * s

        wait_stage(idx1, si1)
        gathers(idx1, rows1, sg1)
        @pl.when(s < NSUP - 1)
        def _():
            stage(l0 + 2 * DBLK, idx0, si0)

        @pl.when(s > 0)
        def _():
            wait_outs(tr0, so0)

        wait_gather1(sg0)
        transpose_add(l0, rows0, 0, tr0.at[0])
        wait_gather1(sg0)
        transpose_add(l0 + 1, rows0, BGRP, tr0.at[1])
        outs(tr0, l0, so0)

        @pl.when(s < NSUP - 1)
        def _():
            wait_stage(idx0, si0)
            gathers(idx0, rows0, sg0)

        @pl.when(s < NSUP - 1)
        def _():
            stage(l0 + 3 * DBLK, idx1, si1)

        @pl.when(s > 0)
        def _():
            wait_outs(tr1, so1)

        wait_gather1(sg1)
        transpose_add(l0 + DBLK, rows1, 0, tr1.at[0])
        wait_gather1(sg1)
        transpose_add(l0 + DBLK + 1, rows1, BGRP, tr1.at[1])
        outs(tr1, l0 + DBLK, so1)
        return carry

    lax.fori_loop(0, NSUP, superstep, 0)
    wait_outs(tr0, so0)
    wait_outs(tr1, so1)


@jax.jit
def _emb(table, pos2d, ids_t):
    mesh = plsc.VectorSubcoreMesh(core_axis_name="c", subcore_axis_name="s")
    return pl.kernel(
        _emb_body,
        out_type=jax.ShapeDtypeStruct((MAX_POS, EMBED // 8, NW, 8, BGRP), jnp.float32),
        mesh=mesh,
        scratch_types=[
            pltpu.VMEM((MAX_POS, EMBED), jnp.float32),
            pltpu.VMEM((DBLK, BGRP), jnp.int32),
            pltpu.VMEM((DBLK, BGRP), jnp.int32),
            pltpu.VMEM((DBLK * BGRP, EMBED), jnp.float32),
            pltpu.VMEM((DBLK * BGRP, EMBED), jnp.float32),
            pltpu.VMEM((DBLK, EMBED // 8, 8, TPAD), jnp.float32),
            pltpu.VMEM((DBLK, EMBED // 8, 8, TPAD), jnp.float32),
            pltpu.SemaphoreType.DMA,
            pltpu.SemaphoreType.DMA,
            pltpu.SemaphoreType.DMA,
            pltpu.SemaphoreType.DMA,
            pltpu.SemaphoreType.DMA,
            pltpu.SemaphoreType.DMA,
        ],
        compiler_params=pltpu.CompilerParams(
            use_tc_tiling_on_sc=False, needs_layout_passes=False
        ),
    )(table, pos2d, ids_t)


def kernel(embedding_table, position_embeds, input_ids):
    ids_t = jnp.transpose(input_ids.astype(jnp.int32))
    pos2d = position_embeds.reshape(MAX_POS, EMBED)
    outp = _emb(embedding_table, pos2d, ids_t)
    return outp.transpose(2, 4, 0, 1, 3).reshape(BATCH, SEQ, EMBED)
